# Initial kernel scaffold; baseline (speedup 1.0000x reference)
import jax
import jax.numpy as jnp
from jax.experimental import pallas as pl


def kernel(a, idx):
    raise NotImplementedError("write your pallas kernel here")


# speedup vs baseline: 4.8095x; 4.8095x over previous
import functools

import jax
import jax.numpy as jnp
from jax import lax
from jax.experimental import pallas as pl
from jax.experimental.pallas import tpu as pltpu
from jax.experimental.pallas import tpu_sc as plsc

_NUM_CORES = 2
_NUM_SUBCORES = 16
_NUM_WORKERS = _NUM_CORES * _NUM_SUBCORES

_CHUNK = 1024


def _gather_kernel(n_rows, d, table_hbm, idx_hbm, out_hbm, idx_v, rows_v, sem):
    rows_per_worker = n_rows // _NUM_WORKERS
    n_chunks = rows_per_worker // _CHUNK
    wid = lax.axis_index("s") * _NUM_CORES + lax.axis_index("c")
    worker_base = wid * rows_per_worker

    def body(c, carry):
        base = worker_base + c * _CHUNK
        pltpu.sync_copy(idx_hbm.at[pl.ds(base, _CHUNK)], idx_v)
        pltpu.async_copy(table_hbm.at[idx_v], rows_v, sem).wait()
        pltpu.sync_copy(rows_v, out_hbm.at[pl.ds(base, _CHUNK)])
        return carry

    lax.fori_loop(0, n_chunks, body, 0, unroll=False)


def kernel(a, idx):
    n_rows = idx.size
    d = a.shape[1]
    idx_flat = idx.reshape(-1).astype(jnp.int32)

    mesh = plsc.VectorSubcoreMesh(
        core_axis_name="c", subcore_axis_name="s",
        num_cores=_NUM_CORES, num_subcores=_NUM_SUBCORES,
    )
    k = pl.kernel(
        functools.partial(_gather_kernel, n_rows, d),
        out_type=jax.ShapeDtypeStruct((n_rows, d), jnp.float32),
        mesh=mesh,
        scratch_types=[
            pltpu.VMEM((_CHUNK,), jnp.int32),
            pltpu.VMEM((_CHUNK, d), jnp.float32),
            pltpu.SemaphoreType.DMA,
        ],
        compiler_params=pltpu.CompilerParams(use_tc_tiling_on_sc=False),
    )
    out = k(a, idx_flat)
    return out.reshape(idx.shape + (d,))


# speedup vs baseline: 5.0461x; 1.0492x over previous
import functools

import jax
import jax.numpy as jnp
from jax import lax
from jax.experimental import pallas as pl
from jax.experimental.pallas import tpu as pltpu
from jax.experimental.pallas import tpu_sc as plsc

_NUM_CORES = 2
_NUM_SUBCORES = 16
_NUM_WORKERS = _NUM_CORES * _NUM_SUBCORES

_CHUNK = 1024


def _gather_kernel(n_rows, d, table_hbm, idx_hbm, out_hbm,
                   idx0, idx1, rows0, rows1, g0, g1, w0, w1):
    rows_per_worker = n_rows // _NUM_WORKERS
    n_chunks = rows_per_worker // _CHUNK
    n_pairs = n_chunks // 2
    wid = lax.axis_index("s") * _NUM_CORES + lax.axis_index("c")
    worker_base = wid * rows_per_worker

    pltpu.sync_copy(idx_hbm.at[pl.ds(worker_base, _CHUNK)], idx0)
    pltpu.async_copy(table_hbm.at[idx0], rows0, g0)

    def body(i, carry):
        base0 = worker_base + (2 * i) * _CHUNK
        base1 = base0 + _CHUNK
        base2 = base1 + _CHUNK

        pltpu.sync_copy(idx_hbm.at[pl.ds(base1, _CHUNK)], idx1)

        @pl.when(i > 0)
        def _():
            pltpu.make_async_copy(rows1, out_hbm.at[pl.ds(base1 - 2 * _CHUNK, _CHUNK)], w1).wait()

        pltpu.async_copy(table_hbm.at[idx1], rows1, g1)

        pltpu.make_async_copy(table_hbm.at[idx0], rows0, g0).wait()
        pltpu.async_copy(rows0, out_hbm.at[pl.ds(base0, _CHUNK)], w0)

        @pl.when(i < n_pairs - 1)
        def _():
            pltpu.sync_copy(idx_hbm.at[pl.ds(base2, _CHUNK)], idx0)

        pltpu.make_async_copy(rows0, out_hbm.at[pl.ds(base0, _CHUNK)], w0).wait()

        @pl.when(i < n_pairs - 1)
        def _():
            pltpu.async_copy(table_hbm.at[idx0], rows0, g0)

        pltpu.make_async_copy(table_hbm.at[idx1], rows1, g1).wait()
        pltpu.async_copy(rows1, out_hbm.at[pl.ds(base1, _CHUNK)], w1)
        return carry

    lax.fori_loop(0, n_pairs, body, 0, unroll=False)

    last_base = worker_base + (n_chunks - 1) * _CHUNK
    pltpu.make_async_copy(rows1, out_hbm.at[pl.ds(last_base, _CHUNK)], w1).wait()


def kernel(a, idx):
    n_rows = idx.size
    d = a.shape[1]
    idx_flat = idx.reshape(-1).astype(jnp.int32)

    mesh = plsc.VectorSubcoreMesh(
        core_axis_name="c", subcore_axis_name="s",
        num_cores=_NUM_CORES, num_subcores=_NUM_SUBCORES,
    )
    k = pl.kernel(
        functools.partial(_gather_kernel, n_rows, d),
        out_type=jax.ShapeDtypeStruct((n_rows, d), jnp.float32),
        mesh=mesh,
        scratch_types=[
            pltpu.VMEM((_CHUNK,), jnp.int32),
            pltpu.VMEM((_CHUNK,), jnp.int32),
            pltpu.VMEM((_CHUNK, d), jnp.float32),
            pltpu.VMEM((_CHUNK, d), jnp.float32),
            pltpu.SemaphoreType.DMA,
            pltpu.SemaphoreType.DMA,
            pltpu.SemaphoreType.DMA,
            pltpu.SemaphoreType.DMA,
        ],
        compiler_params=pltpu.CompilerParams(use_tc_tiling_on_sc=False),
    )
    out = k(a, idx_flat)
    return out.reshape(idx.shape + (d,))


# speedup vs baseline: 5.0518x; 1.0011x over previous
import functools

import jax
import jax.numpy as jnp
from jax import lax
from jax.experimental import pallas as pl
from jax.experimental.pallas import tpu as pltpu
from jax.experimental.pallas import tpu_sc as plsc

_NUM_CORES = 2
_NUM_SUBCORES = 16
_NUM_WORKERS = _NUM_CORES * _NUM_SUBCORES

_CHUNK = 1024
_NSTREAM = 4
_SUB = _CHUNK // _NSTREAM


def _fire_gather(table_hbm, idx_v, rows_v, sem):
    for s in range(_NSTREAM):
        pltpu.async_copy(
            table_hbm.at[idx_v.at[pl.ds(s * _SUB, _SUB)]],
            rows_v.at[pl.ds(s * _SUB, _SUB)], sem)


def _drain_gather(table_hbm, idx_v, rows_v, sem):
    for s in range(_NSTREAM):
        pltpu.make_async_copy(
            table_hbm.at[idx_v.at[pl.ds(s * _SUB, _SUB)]],
            rows_v.at[pl.ds(s * _SUB, _SUB)], sem).wait()


def _gather_kernel(n_rows, d, table_hbm, idx_hbm, out_hbm,
                   idx0, idx1, rows0, rows1, g0, g1, w0, w1):
    rows_per_worker = n_rows // _NUM_WORKERS
    n_chunks = rows_per_worker // _CHUNK
    n_pairs = n_chunks // 2
    wid = lax.axis_index("s") * _NUM_CORES + lax.axis_index("c")
    worker_base = wid * rows_per_worker

    pltpu.sync_copy(idx_hbm.at[pl.ds(worker_base, _CHUNK)], idx0)
    _fire_gather(table_hbm, idx0, rows0, g0)

    def body(i, carry):
        base0 = worker_base + (2 * i) * _CHUNK
        base1 = base0 + _CHUNK
        base2 = base1 + _CHUNK

        pltpu.sync_copy(idx_hbm.at[pl.ds(base1, _CHUNK)], idx1)

        @pl.when(i > 0)
        def _():
            pltpu.make_async_copy(rows1, out_hbm.at[pl.ds(base1 - 2 * _CHUNK, _CHUNK)], w1).wait()

        _fire_gather(table_hbm, idx1, rows1, g1)

        _drain_gather(table_hbm, idx0, rows0, g0)
        pltpu.async_copy(rows0, out_hbm.at[pl.ds(base0, _CHUNK)], w0)

        @pl.when(i < n_pairs - 1)
        def _():
            pltpu.sync_copy(idx_hbm.at[pl.ds(base2, _CHUNK)], idx0)

        pltpu.make_async_copy(rows0, out_hbm.at[pl.ds(base0, _CHUNK)], w0).wait()

        @pl.when(i < n_pairs - 1)
        def _():
            _fire_gather(table_hbm, idx0, rows0, g0)

        _drain_gather(table_hbm, idx1, rows1, g1)
        pltpu.async_copy(rows1, out_hbm.at[pl.ds(base1, _CHUNK)], w1)
        return carry

    lax.fori_loop(0, n_pairs, body, 0, unroll=False)

    last_base = worker_base + (n_chunks - 1) * _CHUNK
    pltpu.make_async_copy(rows1, out_hbm.at[pl.ds(last_base, _CHUNK)], w1).wait()


def kernel(a, idx):
    n_rows = idx.size
    d = a.shape[1]
    idx_flat = idx.reshape(-1).astype(jnp.int32)

    mesh = plsc.VectorSubcoreMesh(
        core_axis_name="c", subcore_axis_name="s",
        num_cores=_NUM_CORES, num_subcores=_NUM_SUBCORES,
    )
    k = pl.kernel(
        functools.partial(_gather_kernel, n_rows, d),
        out_type=jax.ShapeDtypeStruct((n_rows, d), jnp.float32),
        mesh=mesh,
        scratch_types=[
            pltpu.VMEM((_CHUNK,), jnp.int32),
            pltpu.VMEM((_CHUNK,), jnp.int32),
            pltpu.VMEM((_CHUNK, d), jnp.float32),
            pltpu.VMEM((_CHUNK, d), jnp.float32),
            pltpu.SemaphoreType.DMA,
            pltpu.SemaphoreType.DMA,
            pltpu.SemaphoreType.DMA,
            pltpu.SemaphoreType.DMA,
        ],
        compiler_params=pltpu.CompilerParams(use_tc_tiling_on_sc=False),
    )
    out = k(a, idx_flat)
    return out.reshape(idx.shape + (d,))
